# baseline (speedup 1.0000x reference)
import functools

import jax
import jax.numpy as jnp
from jax import lax
from jax.experimental import pallas as pl
from jax.experimental.pallas import tpu as pltpu
from jax.experimental.pallas import tpu_sc as plsc

VOCAB = 100000
DIM = 50
CTX = 200

ROWS_PER_TILE = 8
N_ACTIVE = CTX // ROWS_PER_TILE

BLK = 2000
NBLK = VOCAB // BLK


def _sc_gather(idx, table):
    info = plsc.get_sparse_core_info()
    nc = info.num_cores

    mesh = plsc.VectorSubcoreMesh(core_axis_name="c", subcore_axis_name="s")

    @functools.partial(
        pl.kernel,
        mesh=mesh,
        compiler_params=pltpu.CompilerParams(use_tc_tiling_on_sc=False),
        out_type=jax.ShapeDtypeStruct((CTX, DIM), jnp.float32),
        scratch_types=[
            pltpu.VMEM((ROWS_PER_TILE,), jnp.int32),
            pltpu.VMEM((ROWS_PER_TILE, DIM), jnp.float32),
            pltpu.SemaphoreType.DMA,
        ],
    )
    def gather_kernel(idx_hbm, table_hbm, out_hbm, idx_v, rows_v, sem):
        wid = lax.axis_index("s") * nc + lax.axis_index("c")

        @pl.when(wid < N_ACTIVE)
        def _():
            base = wid * ROWS_PER_TILE
            pltpu.sync_copy(idx_hbm.at[pl.ds(base, ROWS_PER_TILE)], idx_v)
            pltpu.async_copy(table_hbm.at[idx_v], rows_v, sem).wait()
            pltpu.sync_copy(rows_v, out_hbm.at[pl.ds(base, ROWS_PER_TILE)])

    return gather_kernel(idx, table)


def _tc_head_body(g_ref, w_ref, b_ref, out_ref, pooled, m_ref, s_ref):
    i = pl.program_id(0)

    @pl.when(i == 0)
    def _():
        pooled[...] = jnp.sum(g_ref[...], axis=0, keepdims=True) * (1.0 / CTX)
        m_ref[0] = -jnp.inf
        s_ref[0] = 0.0

    logits = lax.dot_general(
        pooled[...], w_ref[...],
        (((1,), (1,)), ((), ())),
        preferred_element_type=jnp.float32,
        precision=lax.Precision.HIGHEST,
    ) + b_ref[0]
    out_ref[pl.ds(i, 1)] = logits.reshape(1, 1, BLK)

    m_old = m_ref[0]
    m_new = jnp.maximum(m_old, jnp.max(logits))
    s_ref[0] = s_ref[0] * jnp.exp(m_old - m_new) + jnp.sum(
        jnp.exp(logits - m_new))
    m_ref[0] = m_new

    @pl.when(i == NBLK - 1)
    def _():
        lse = m_ref[0] + jnp.log(s_ref[0])
        out_ref[...] = out_ref[...] - lse


def _tc_head(gathered, W, b3d, interpret=False):
    return pl.pallas_call(
        _tc_head_body,
        grid=(NBLK,),
        in_specs=[
            pl.BlockSpec((CTX, DIM), lambda i: (0, 0)),
            pl.BlockSpec((BLK, DIM), lambda i: (i, 0)),
            pl.BlockSpec((1, 1, BLK), lambda i: (i, 0, 0)),
        ],
        out_specs=pl.BlockSpec((NBLK, 1, BLK), lambda i: (0, 0, 0)),
        out_shape=jax.ShapeDtypeStruct((NBLK, 1, BLK), jnp.float32),
        scratch_shapes=[
            pltpu.VMEM((1, DIM), jnp.float32),
            pltpu.SMEM((1,), jnp.float32),
            pltpu.SMEM((1,), jnp.float32),
        ],
        interpret=interpret,
    )(gathered, W, b3d)


def kernel(inputs, table, W, b):
    idx = inputs.astype(jnp.int32)
    gathered = _sc_gather(idx, table)
    out = _tc_head(gathered, W, b.reshape(NBLK, 1, BLK))
    return out.reshape(1, VOCAB)


# speedup vs baseline: 2.2023x; 2.2023x over previous
import functools

import jax
import jax.numpy as jnp
from jax import lax
from jax.experimental import pallas as pl
from jax.experimental.pallas import tpu as pltpu
from jax.experimental.pallas import tpu_sc as plsc

VOCAB = 100000
DIM = 50
CTX = 200

ROWS_PER_TILE = 8
N_ACTIVE = CTX // ROWS_PER_TILE

BLK = 10000
NBLK = VOCAB // BLK


def _sc_gather(idx, table):
    info = plsc.get_sparse_core_info()
    nc = info.num_cores

    mesh = plsc.VectorSubcoreMesh(core_axis_name="c", subcore_axis_name="s")

    @functools.partial(
        pl.kernel,
        mesh=mesh,
        compiler_params=pltpu.CompilerParams(needs_layout_passes=False),
        out_type=jax.ShapeDtypeStruct((CTX, DIM), jnp.float32),
        scratch_types=[
            pltpu.VMEM((16,), jnp.int32),
            pltpu.VMEM((ROWS_PER_TILE, DIM), jnp.float32),
            pltpu.SemaphoreType.DMA,
        ],
    )
    def gather_kernel(idx_hbm, table_hbm, out_hbm, idx_v, rows_v, sem):
        wid = lax.axis_index("s") * nc + lax.axis_index("c")

        @pl.when(wid < N_ACTIVE)
        def _():
            base = wid * ROWS_PER_TILE
            pltpu.sync_copy(idx_hbm.at[pl.ds(base, ROWS_PER_TILE)],
                            idx_v.at[pl.ds(0, ROWS_PER_TILE)])
            lane = lax.iota(jnp.int32, 16)
            idxs = idx_v[...]
            copies = []
            for k in range(ROWS_PER_TILE):
                r = jnp.max(jnp.where(lane == k, idxs, 0))
                copies.append(pltpu.async_copy(
                    table_hbm.at[pl.ds(r, 1)], rows_v.at[pl.ds(k, 1)], sem))
            for c in copies:
                c.wait()
            pltpu.sync_copy(rows_v, out_hbm.at[pl.ds(base, ROWS_PER_TILE)])

    return gather_kernel(idx, table)


def _tc_head_body(g_ref, w_ref, b_ref, out_ref, pooled, m_ref):
    i = pl.program_id(0)

    @pl.when(i == 0)
    def _():
        pooled[...] = jnp.sum(g_ref[...], axis=0, keepdims=True) * (1.0 / CTX)
        m_ref[0] = -jnp.inf

    logits = lax.dot_general(
        pooled[...], w_ref[...],
        (((1,), (1,)), ((), ())),
        preferred_element_type=jnp.float32,
    ) + b_ref[0]
    out_ref[pl.ds(i, 1), :] = logits
    m_ref[0] = jnp.maximum(m_ref[0], jnp.max(logits))

    @pl.when(i == NBLK - 1)
    def _():
        m = m_ref[0]
        lse = m + jnp.log(jnp.sum(jnp.exp(out_ref[...] - m)))
        out_ref[...] = out_ref[...] - lse


def _tc_head(gathered, W, b2d, interpret=False):
    return pl.pallas_call(
        _tc_head_body,
        grid=(NBLK,),
        in_specs=[
            pl.BlockSpec((CTX, DIM), lambda i: (0, 0)),
            pl.BlockSpec((BLK, DIM), lambda i: (i, 0)),
            pl.BlockSpec((1, 1, BLK), lambda i: (i, 0, 0)),
        ],
        out_specs=pl.BlockSpec((NBLK, BLK), lambda i: (0, 0)),
        out_shape=jax.ShapeDtypeStruct((NBLK, BLK), jnp.float32),
        scratch_shapes=[
            pltpu.VMEM((1, DIM), jnp.float32),
            pltpu.SMEM((1,), jnp.float32),
        ],
        compiler_params=pltpu.CompilerParams(
            dimension_semantics=("arbitrary",)),
        interpret=interpret,
    )(gathered, W, b2d)


def kernel(inputs, table, W, b):
    idx = inputs.astype(jnp.int32)
    gathered = _sc_gather(idx, table)
    out = _tc_head(gathered, W, b.reshape(NBLK, 1, BLK))
    return out.reshape(1, VOCAB)
